# baseline (speedup 1.0000x reference)
import jax
import jax.numpy as jnp
from jax import lax
from jax.experimental import pallas as pl
from jax.experimental.pallas import tpu as pltpu
from jax.experimental.pallas import tpu_sc as plsc

EMBED = 64
PADDED_W = 128
BATCH = 16384
N_TABLE = 100000
NC = 2
NS = 16
NW = NC * NS
B_PER_W = BATCH // NW
IDX_CHUNK = 128
N_CHUNKS = B_PER_W // IDX_CHUNK
GROUPS_PER_CHUNK = IDX_CHUNK // 16

PACK_BLK = 10000
N_PACK_BLKS = N_TABLE // PACK_BLK


HALF_T = N_TABLE // 2
PACK_ROWS = PACK_BLK // 2
N_HALF_BLKS = HALF_T // PACK_ROWS


def _tc_pack_body(a_ref, b_ref, o_ref):
    o_ref[:, :EMBED] = a_ref[:, :EMBED]
    o_ref[:, EMBED:] = b_ref[:, :EMBED]


def _tc_pack(table):
    return pl.pallas_call(
        _tc_pack_body,
        grid=(N_HALF_BLKS,),
        in_specs=[
            pl.BlockSpec((PACK_ROWS, 65), lambda i: (i, 0)),
            pl.BlockSpec((PACK_ROWS, 65), lambda i: (i + N_HALF_BLKS, 0)),
        ],
        out_specs=pl.BlockSpec((PACK_ROWS, PADDED_W), lambda i: (i, 0)),
        out_shape=jax.ShapeDtypeStruct((HALF_T, PADDED_W), jnp.float32),
    )(table, table)


def _sc_body(gidx_hbm, par_hbm, table_hbm, ent_hbm, out_hbm,
             idx_v, par_v, rows_v, ent_v, pad_e, pad_o, out_v,
             sg0, sg1, sent):
    wid = lax.axis_index("s") * NC + lax.axis_index("c")
    base = wid * B_PER_W
    gsems = [sg0, sg1]

    pltpu.sync_copy(gidx_hbm.at[wid], idx_v)
    pltpu.sync_copy(par_hbm.at[wid], par_v)

    def start_gather(j):
        return pltpu.async_copy(
            table_hbm.at[idx_v.at[j]], rows_v.at[j % 2], gsems[j % 2])

    gathers = [start_gather(0), start_gather(1)]
    ecopy = pltpu.async_copy(ent_hbm.at[pl.ds(base, B_PER_W)], ent_v, sent)
    ecopy.wait()

    lanes = lax.iota(jnp.int32, 16)

    for j in range(N_CHUNKS):
        gathers[j].wait()
        buf = rows_v.at[j % 2]
        ebuf = ent_v.at[pl.ds(j * IDX_CHUNK, IDX_CHUNK)]

        def group_body(g, _, j=j, buf=buf, ebuf=ebuf):
            row0 = g * 16
            for r in range(16):
                row = row0 + r
                e0 = ebuf[row, pl.ds(0, 16)]
                acc_e = buf[row, pl.ds(0, 16)] * e0
                acc_o = buf[row, pl.ds(EMBED, 16)] * e0
                for q in range(1, 4):
                    eq = ebuf[row, pl.ds(16 * q, 16)]
                    acc_e = acc_e + buf[row, pl.ds(16 * q, 16)] * eq
                    acc_o = acc_o + buf[row, pl.ds(EMBED + 16 * q, 16)] * eq
                pad_e[r, pl.ds(0, 16)] = acc_e
                pad_o[r, pl.ds(0, 16)] = acc_o
            tot_e = plsc.load_gather(
                pad_e, [lanes, jnp.full((16,), 0, jnp.int32)])
            tot_o = plsc.load_gather(
                pad_o, [lanes, jnp.full((16,), 0, jnp.int32)])
            for c in range(1, 16):
                cc = jnp.full((16,), c, jnp.int32)
                tot_e = tot_e + plsc.load_gather(pad_e, [lanes, cc])
                tot_o = tot_o + plsc.load_gather(pad_o, [lanes, cc])
            par = par_v[j, pl.ds(row0, 16)]
            parf = par.astype(jnp.float32)
            tot = tot_e + parf * (tot_o - tot_e)
            bias = plsc.load_gather(buf, [row0 + lanes, par * EMBED + 1])
            sig = 1.0 / (1.0 + jnp.exp(-(tot + bias)))
            out_v[pl.ds(row0, 16)] = sig
            return 0

        lax.fori_loop(0, GROUPS_PER_CHUNK, group_body, 0)
        pltpu.sync_copy(
            out_v, out_hbm.at[pl.ds(base + j * IDX_CHUNK, IDX_CHUNK)])
        if j + 2 < N_CHUNKS:
            gathers.append(start_gather(j + 2))


def _sc_fused(gidx, par, table_packed, ent):
    mesh = plsc.VectorSubcoreMesh(core_axis_name="c", subcore_axis_name="s")
    kern = pl.kernel(
        _sc_body,
        mesh=mesh,
        out_type=jax.ShapeDtypeStruct((BATCH,), jnp.float32),
        scratch_types=[
            pltpu.VMEM((N_CHUNKS, IDX_CHUNK), jnp.int32),
            pltpu.VMEM((N_CHUNKS, IDX_CHUNK), jnp.int32),
            pltpu.VMEM((2, IDX_CHUNK, PADDED_W), jnp.float32),
            pltpu.VMEM((B_PER_W, EMBED), jnp.float32),
            pltpu.VMEM((16, 17), jnp.float32),
            pltpu.VMEM((16, 17), jnp.float32),
            pltpu.VMEM((IDX_CHUNK,), jnp.float32),
            pltpu.SemaphoreType.DMA,
            pltpu.SemaphoreType.DMA,
            pltpu.SemaphoreType.DMA,
        ],
        compiler_params=pltpu.CompilerParams(needs_layout_passes=False),
    )
    return kern(
        gidx.reshape(NW, N_CHUNKS, IDX_CHUNK),
        par.reshape(NW, N_CHUNKS, IDX_CHUNK),
        table_packed, ent)


def kernel(ent, att, att_embed_weight):
    att = att.astype(jnp.int32)
    table_packed = _tc_pack(att_embed_weight)
    par = (att >= HALF_T).astype(jnp.int32)
    gidx = att - par * HALF_T
    return _sc_fused(gidx, par, table_packed, ent)


# speedup vs baseline: 1.1576x; 1.1576x over previous
import jax
import jax.numpy as jnp
from jax import lax
from jax.experimental import pallas as pl
from jax.experimental.pallas import tpu as pltpu
from jax.experimental.pallas import tpu_sc as plsc

EMBED = 64
BATCH = 16384
N_TABLE = 100000
TABLE_W = 65
NC = 2
NS = 16
NW = NC * NS
B_PER_W = BATCH // NW
N_GROUPS = B_PER_W // 16
NBUF = 32


def _sc_body(att_hbm, table_hbm, ent_hbm, out_hbm,
             idx_v, t8_v, ent_v, pad_v, out_v, sg, sent):
    wid = lax.axis_index("s") * NC + lax.axis_index("c")
    base = wid * B_PER_W

    pltpu.sync_copy(att_hbm.at[wid], idx_v)
    ecopy = pltpu.async_copy(ent_hbm.at[pl.ds(base, B_PER_W)], ent_v, sent)

    def issue_group(g):
        tv = idx_v[pl.ds(g * 16, 16)]
        for r in range(16):
            i = g * 16 + r
            t = tv[r]
            tb = pl.multiple_of(t - jnp.bitwise_and(t, 7), 8)
            pltpu.async_copy(
                table_hbm.at[pl.ds(tb, 8)],
                t8_v.at[jnp.bitwise_and(i, NBUF - 1)],
                sg,
            )

    issue_group(0)
    issue_group(1)
    ecopy.wait()

    lanes = lax.iota(jnp.int32, 16)

    def group_body(g, _):
        for r in range(16):
            pltpu.make_async_copy(
                table_hbm.at[pl.ds(0, 8)], t8_v.at[0], sg).wait()
        row0 = g * 16
        tv = idx_v[pl.ds(row0, 16)]
        for r in range(16):
            i = row0 + r
            t = tv[r]
            sub = jnp.bitwise_and(t, 7)
            buf = jnp.bitwise_and(i, NBUF - 1)
            acc = t8_v[buf, sub, pl.ds(0, 16)] * ent_v[i, pl.ds(0, 16)]
            for q in range(1, 4):
                acc = acc + (t8_v[buf, sub, pl.ds(16 * q, 16)]
                             * ent_v[i, pl.ds(16 * q, 16)])
            pad_v[r, pl.ds(0, 16)] = acc
        tot = plsc.load_gather(pad_v, [lanes, jnp.full((16,), 0, jnp.int32)])
        for c in range(1, 16):
            tot = tot + plsc.load_gather(
                pad_v, [lanes, jnp.full((16,), c, jnp.int32)])
        buf_vec = jnp.bitwise_and(row0 + lanes, NBUF - 1)
        sub_vec = jnp.bitwise_and(tv, 7)
        bias = plsc.load_gather(
            t8_v, [buf_vec, sub_vec, jnp.full((16,), 1, jnp.int32)])
        sig = 1.0 / (1.0 + jnp.exp(-(tot + bias)))
        out_v[pl.ds(row0, 16)] = sig

        @pl.when(g < N_GROUPS - 2)
        def _():
            issue_group(g + 2)

        return 0

    lax.fori_loop(0, N_GROUPS, group_body, 0)
    pltpu.sync_copy(out_v, out_hbm.at[pl.ds(base, B_PER_W)])


def kernel(ent, att, att_embed_weight):
    att = att.astype(jnp.int32)
    mesh = plsc.VectorSubcoreMesh(core_axis_name="c", subcore_axis_name="s")
    kern = pl.kernel(
        _sc_body,
        mesh=mesh,
        out_type=jax.ShapeDtypeStruct((BATCH,), jnp.float32),
        scratch_types=[
            pltpu.VMEM((B_PER_W,), jnp.int32),
            pltpu.VMEM((NBUF, 8, TABLE_W), jnp.float32),
            pltpu.VMEM((B_PER_W, EMBED), jnp.float32),
            pltpu.VMEM((16, 17), jnp.float32),
            pltpu.VMEM((B_PER_W,), jnp.float32),
            pltpu.SemaphoreType.DMA,
            pltpu.SemaphoreType.DMA,
        ],
        compiler_params=pltpu.CompilerParams(needs_layout_passes=False),
    )
    return kern(att.reshape(NW, B_PER_W), att_embed_weight, ent)
